# baseline (speedup 1.0000x reference)
import functools
import math

import jax
import jax.numpy as jnp
from jax.experimental import pallas as pl
from jax.experimental.pallas import tpu as pltpu

N = 10000
E = 160000
H = 8
FH = 32
HID = 256


def _mm_kernel(x_ref, w_ref, b_ref, o_ref, *, act, ln, g_ref=None, be_ref=None):
    acc = jnp.dot(x_ref[...], w_ref[...], preferred_element_type=jnp.float32)
    acc = acc + b_ref[...]
    if ln:
        m = jnp.mean(acc, axis=-1, keepdims=True)
        v = jnp.mean((acc - m) ** 2, axis=-1, keepdims=True)
        acc = (acc - m) * jax.lax.rsqrt(v + 1e-5) * g_ref[...] + be_ref[...]
    if act == "relu":
        acc = jnp.maximum(acc, 0.0)
    o_ref[...] = acc


def _mm(x, W, b, g=None, be=None, act=None, tm=512):
    M, K = x.shape
    N_out = W.shape[1]
    Mp = math.ceil(M / tm) * tm
    if Mp != M:
        x = jnp.pad(x, ((0, Mp - M), (0, 0)))
    ln = g is not None
    kern = functools.partial(_mm_kernel, act=act, ln=ln)
    in_specs = [
        pl.BlockSpec((tm, K), lambda i: (i, 0)),
        pl.BlockSpec((K, N_out), lambda i: (0, 0)),
        pl.BlockSpec((N_out,), lambda i: (0,)),
    ]
    args = [x, W, b]
    if ln:
        def kern(x_ref, w_ref, b_ref, g_ref, be_ref, o_ref):
            _mm_kernel(x_ref, w_ref, b_ref, o_ref, act=act, ln=True,
                       g_ref=g_ref, be_ref=be_ref)
        in_specs += [
            pl.BlockSpec((N_out,), lambda i: (0,)),
            pl.BlockSpec((N_out,), lambda i: (0,)),
        ]
        args += [g, be]
    out = pl.pallas_call(
        kern,
        grid=(Mp // tm,),
        in_specs=in_specs,
        out_specs=pl.BlockSpec((tm, N_out), lambda i: (i, 0)),
        out_shape=jax.ShapeDtypeStruct((Mp, N_out), jnp.float32),
    )(*args)
    return out[:M]


def _ln(x, g, b):
    m = jnp.mean(x, axis=-1, keepdims=True)
    v = jnp.var(x, axis=-1, keepdims=True)
    return (x - m) / jnp.sqrt(v + 1e-5) * g + b


def _gatv2(x, src2, dst2, e, Wl, bl, Wr, br, att, bias):
    n = x.shape[0]
    x_l = _mm(x, Wl, bl).reshape(n, H, FH)
    x_r = _mm(x, Wr, br).reshape(n, H, FH)
    m = x_l[src2] + x_r[dst2] + e.reshape(-1, H, FH)
    m = jax.nn.leaky_relu(m, 0.2)
    alpha = jnp.sum(m * att[None], axis=-1)
    amax = jnp.full((n, H), -jnp.inf, x.dtype).at[dst2].max(alpha)
    ex = jnp.exp(alpha - amax[dst2])
    denom = jnp.zeros((n, H), x.dtype).at[dst2].add(ex)
    a = ex / (denom[dst2] + 1e-16)
    out = jnp.zeros((n, H, FH), x.dtype).at[dst2].add(x_l[src2] * a[..., None])
    return out.reshape(n, H * FH) + bias


def kernel(x, edge_index, edge_attr, category, class_tab, index_tab, np_W, np_b, np_g, np_be, ep_W, ep_b, ep_g, ep_be, l0_Wl, l0_bl, l0_Wr, l0_br, l0_We, l0_att, l0_bias, l0_g, l0_be, l1_Wl, l1_bl, l1_Wr, l1_br, l1_We, l1_att, l1_bias, l1_g, l1_be, l2_Wl, l2_bl, l2_Wr, l2_br, l2_We, l2_att, l2_bias, l2_g, l2_be, th_W1, th_b1, th_W2, th_b2, rh_W1, rh_b1, rh_W2, rh_b2):
    src, dst = edge_index[0], edge_index[1]
    cls = class_tab[category]
    bidx = jnp.arange(N) % 50
    idx_emb = index_tab[bidx]
    anchor = jnp.zeros((N, 1), jnp.float32).at[0, 0].set(1.0)
    h0 = jnp.concatenate([cls, x, idx_emb, anchor], axis=1)
    h = _mm(h0, np_W, np_b, np_g, np_be, act="relu")
    ea = _mm(edge_attr, ep_W, ep_b, ep_g, ep_be, act="relu")

    deg = jnp.zeros((N,), jnp.float32).at[dst].add(1.0)
    ea_mean = jnp.zeros((N, HID), jnp.float32).at[dst].add(ea) / jnp.maximum(deg, 1.0)[:, None]
    loop = jnp.arange(N)
    src2 = jnp.concatenate([src, loop])
    dst2 = jnp.concatenate([dst, loop])
    ea2 = jnp.concatenate([ea, ea_mean], axis=0)

    layers = [
        (l0_Wl, l0_bl, l0_Wr, l0_br, l0_We, l0_att, l0_bias, l0_g, l0_be),
        (l1_Wl, l1_bl, l1_Wr, l1_br, l1_We, l1_att, l1_bias, l1_g, l1_be),
        (l2_Wl, l2_bl, l2_Wr, l2_br, l2_We, l2_att, l2_bias, l2_g, l2_be),
    ]
    for (Wl, bl, Wr, br, We, att, bias, g, be) in layers:
        e = _mm(ea2, We, jnp.zeros((HID,), jnp.float32))
        h_in = h
        h = _gatv2(h, src2, dst2, e, Wl, bl, Wr, br, att, bias)
        h = h + h_in
        h = _ln(h, g, be)
        h = jax.nn.elu(h)

    raw = jnp.tanh(_mm(h, th_W1, th_b1, act="relu") @ th_W2 + th_b2)
    pos = raw * 2.0
    q = _mm(h, rh_W1, rh_b1, act="relu") @ rh_W2 + rh_b2
    q = q / jnp.maximum(jnp.linalg.norm(q, axis=-1, keepdims=True), 1e-12)
    pos = pos.at[0].set(0.0)
    return pos, q


# speedup vs baseline: 5.4435x; 5.4435x over previous
import functools
import math

import jax
import jax.numpy as jnp
from jax import lax
from jax.experimental import pallas as pl
from jax.experimental.pallas import tpu as pltpu
from jax.experimental.pallas import tpu_sc as plsc

N = 10000
E = 160000
H = 8
FH = 32
HID = 256

NSUB = 16
LANES = 16
EB = 128
NP = 10240
NROW = NP // NSUB
ZCH = 128

_SC_MESH = plsc.VectorSubcoreMesh(
    core_axis_name="c", subcore_axis_name="s", num_cores=2, num_subcores=16)


def _mm_kernel(x_ref, w_ref, b_ref, o_ref, *, act, ln, g_ref=None, be_ref=None):
    acc = jnp.dot(x_ref[...], w_ref[...], preferred_element_type=jnp.float32)
    acc = acc + b_ref[...]
    if ln:
        m = jnp.mean(acc, axis=-1, keepdims=True)
        v = jnp.mean((acc - m) ** 2, axis=-1, keepdims=True)
        acc = (acc - m) * lax.rsqrt(v + 1e-5) * g_ref[...] + be_ref[...]
    if act == "relu":
        acc = jnp.maximum(acc, 0.0)
    o_ref[...] = acc


def _mm(x, W, b, g=None, be=None, act=None, tm=512):
    M, K = x.shape
    N_out = W.shape[1]
    Mp = math.ceil(M / tm) * tm
    if Mp != M:
        x = jnp.pad(x, ((0, Mp - M), (0, 0)))
    ln = g is not None
    kern = functools.partial(_mm_kernel, act=act, ln=False)
    in_specs = [
        pl.BlockSpec((tm, K), lambda i: (i, 0)),
        pl.BlockSpec((K, N_out), lambda i: (0, 0)),
        pl.BlockSpec((N_out,), lambda i: (0,)),
    ]
    args = [x, W, b]
    if ln:
        def kern(x_ref, w_ref, b_ref, g_ref, be_ref, o_ref):
            _mm_kernel(x_ref, w_ref, b_ref, o_ref, act=act, ln=True,
                       g_ref=g_ref, be_ref=be_ref)
        in_specs += [
            pl.BlockSpec((N_out,), lambda i: (0,)),
            pl.BlockSpec((N_out,), lambda i: (0,)),
        ]
        args += [g, be]
    out = pl.pallas_call(
        kern,
        grid=(Mp // tm,),
        in_specs=in_specs,
        out_specs=pl.BlockSpec((tm, N_out), lambda i: (i, 0)),
        out_shape=jax.ShapeDtypeStruct((Mp, N_out), jnp.float32),
    )(*args)
    return out[:M]


def _sc_scatter_body(vcat_hbm, gcat_hbm, wsel_hbm, dst_hbm, out_hbm,
                     acc_sh, gidx_v, dst_v, w_v, rows_v, sem,
                     *, nblk):
    c = lax.axis_index("c")
    s = lax.axis_index("s")
    mtot = nblk * EB * NSUB

    def zrow_body(i, _):
        for j8 in range(8):
            rows_v[i, pl.ds(j8 * 16, 16)] = jnp.zeros((16,), jnp.float32)
        return 0
    lax.fori_loop(0, ZCH, zrow_body, 0)

    def zcopy(z, _):
        r0 = s * NROW + z * ZCH
        pltpu.sync_copy(rows_v, acc_sh.at[pl.ds(r0, ZCH)])
        return 0
    lax.fori_loop(0, NROW // ZCH, zcopy, 0)
    plsc.subcore_barrier()

    mw = nblk * EB
    gdn = lax.GatherDimensionNumbers(
        offset_dims=(), collapsed_slice_dims=(0,), start_index_map=(0,))

    def blk_body(b, _):
        base = s * mw + b * EB
        pltpu.sync_copy(gcat_hbm.at[pl.ds(c * mtot + base, EB)], gidx_v)
        pltpu.sync_copy(dst_hbm.at[pl.ds(base, EB)], dst_v)
        pltpu.sync_copy(wsel_hbm.at[pl.ds(c * mtot * 8 + base * 8, EB * 8)], w_v)
        pltpu.async_copy(vcat_hbm.at[gidx_v], rows_v, sem).wait()

        def ebody(i, _):
            wvec = w_v[pl.ds(i * 16, 16)]
            for half in range(2):
                e = i * 2 + half
                for j in range(4):
                    spl = lax.gather(
                        wvec, jnp.full((16, 1), half * 8 + j, jnp.int32), gdn,
                        slice_sizes=(1,),
                        mode=lax.GatherScatterMode.PROMISE_IN_BOUNDS)
                    for t in range(2):
                        v = j * 2 + t
                        rows_v[e, pl.ds(v * 16, 16)] = (
                            rows_v[e, pl.ds(v * 16, 16)] * spl)
            return 0
        lax.fori_loop(0, EB // 2, ebody, 0)

        pltpu.sync_copy(rows_v, acc_sh.at[dst_v], add=True)
        return 0

    lax.fori_loop(0, nblk, blk_body, 0)
    plsc.subcore_barrier()

    def ecopy(z, _):
        r0 = s * NROW + z * ZCH
        pltpu.sync_copy(acc_sh.at[pl.ds(r0, ZCH)], rows_v)
        pltpu.sync_copy(rows_v, out_hbm.at[c, pl.ds(r0, ZCH)])
        return 0
    lax.fori_loop(0, NROW // ZCH, ecopy, 0)


def _sc_den_body(wflat_hbm, dst_hbm, den_hbm, den_sh, dst_v, wtmp_v, w128_v,
                 *, nblk):
    c = lax.axis_index("c")
    s = lax.axis_index("s")

    def zrow_body(i, _):
        for j8 in range(8):
            w128_v[i, pl.ds(j8 * 16, 16)] = jnp.zeros((16,), jnp.float32)
        return 0
    lax.fori_loop(0, EB, zrow_body, 0)

    def zcopy(z, _):
        r0 = s * NROW + z * ZCH
        pltpu.sync_copy(w128_v, den_sh.at[pl.ds(r0, ZCH)])
        return 0
    lax.fori_loop(0, NROW // ZCH, zcopy, 0)
    plsc.subcore_barrier()

    mw = nblk * EB // 2
    gdn = lax.GatherDimensionNumbers(
        offset_dims=(), collapsed_slice_dims=(0,), start_index_map=(0,))
    hi_idx = ((lax.iota(jnp.int32, 16) % 8) + 8).reshape(16, 1)

    def blk_body(b, _):
        base = (s * 2 + c) * mw + b * EB
        pltpu.sync_copy(dst_hbm.at[pl.ds(base, EB)], dst_v)
        pltpu.sync_copy(wflat_hbm.at[pl.ds(base * 8, EB * 8)], wtmp_v)

        def ebody(i, _):
            wvec = wtmp_v[pl.ds(i * 16, 16)]
            w128_v[i * 2, pl.ds(0, 16)] = wvec
            w128_v[i * 2 + 1, pl.ds(0, 16)] = lax.gather(
                wvec, hi_idx, gdn, slice_sizes=(1,),
                mode=lax.GatherScatterMode.PROMISE_IN_BOUNDS)
            return 0
        lax.fori_loop(0, EB // 2, ebody, 0)

        pltpu.sync_copy(w128_v, den_sh.at[dst_v], add=True)
        return 0
    lax.fori_loop(0, nblk // 2, blk_body, 0)
    plsc.subcore_barrier()

    def ecopy(z, _):
        r0 = s * NROW + z * ZCH
        pltpu.sync_copy(den_sh.at[pl.ds(r0, ZCH)], w128_v)
        pltpu.sync_copy(w128_v, den_hbm.at[c, pl.ds(r0, ZCH)])
        return 0
    lax.fori_loop(0, NROW // ZCH, ecopy, 0)


def _sc_scatter(vals, gidx, wh, dstidx):
    M = gidx.shape[0]
    NV = vals.shape[0]
    nblk = M // (NSUB * EB)
    vcat = jnp.concatenate([vals[:, :128], vals[:, 128:]], axis=0)
    gcat = jnp.concatenate([gidx, gidx + NV])
    wsel = jnp.stack(
        [jnp.pad(wh[:, :4], ((0, 0), (0, 4))),
         jnp.pad(wh[:, 4:], ((0, 0), (0, 4)))]).reshape(-1)
    wpk = wh.reshape(-1)
    acc_fn = pl.kernel(
        functools.partial(_sc_scatter_body, nblk=nblk),
        out_type=jax.ShapeDtypeStruct((2, NP, 128), jnp.float32),
        mesh=_SC_MESH,
        scratch_types=[
            pltpu.VMEM_SHARED((NP, 128), jnp.float32),
            pltpu.VMEM((EB,), jnp.int32),
            pltpu.VMEM((EB,), jnp.int32),
            pltpu.VMEM((EB * 8,), jnp.float32),
            pltpu.VMEM((EB, 128), jnp.float32),
            pltpu.SemaphoreType.DMA,
        ],
    )
    den_fn = pl.kernel(
        functools.partial(_sc_den_body, nblk=nblk),
        out_type=jax.ShapeDtypeStruct((2, NP, 128), jnp.float32),
        mesh=_SC_MESH,
        scratch_types=[
            pltpu.VMEM_SHARED((NP, 128), jnp.float32),
            pltpu.VMEM((EB,), jnp.int32),
            pltpu.VMEM((EB * 8,), jnp.float32),
            pltpu.VMEM((EB, 128), jnp.float32),
        ],
    )
    out = acc_fn(vcat, gcat, wsel, dstidx)
    den2 = den_fn(wpk, dstidx)
    return out, den2[0, :, :8] + den2[1, :, :8]


def _pad_edges(gidx, wh, dstidx):
    M = gidx.shape[0]
    Mp = math.ceil(M / (2 * NSUB * EB)) * (2 * NSUB * EB)
    pad = Mp - M
    gidx = jnp.pad(gidx, (0, pad))
    dstidx = jnp.pad(dstidx, (0, pad))
    wh = jnp.pad(wh, ((0, pad), (0, 0)))
    return gidx, wh, dstidx


def _norm_kernel(lo_ref, hi_ref, den_ref, hin_ref, bias_ref, g_ref, be_ref, o_ref):
    den = den_ref[...]
    tm = den.shape[0]
    den = jnp.broadcast_to(den[:, :, None], (tm, 8, 32)).reshape(tm, 256)
    unnorm = jnp.concatenate([lo_ref[...], hi_ref[...]], axis=1)
    h = unnorm / (den + 1e-16) + bias_ref[...] + hin_ref[...]
    m = jnp.mean(h, axis=-1, keepdims=True)
    v = jnp.mean((h - m) ** 2, axis=-1, keepdims=True)
    h = (h - m) * lax.rsqrt(v + 1e-5) * g_ref[...] + be_ref[...]
    o_ref[...] = jnp.where(h > 0, h, jnp.exp(h) - 1.0)


def _norm_layer(lo, hi, den, h_in, bias, g, be, tm=400):
    grid = (N // tm,)
    return pl.pallas_call(
        _norm_kernel,
        grid=grid,
        in_specs=[
            pl.BlockSpec((tm, 128), lambda i: (i, 0)),
            pl.BlockSpec((tm, 128), lambda i: (i, 0)),
            pl.BlockSpec((tm, 8), lambda i: (i, 0)),
            pl.BlockSpec((tm, 256), lambda i: (i, 0)),
            pl.BlockSpec((256,), lambda i: (0,)),
            pl.BlockSpec((256,), lambda i: (0,)),
            pl.BlockSpec((256,), lambda i: (0,)),
        ],
        out_specs=pl.BlockSpec((tm, 256), lambda i: (i, 0)),
        out_shape=jax.ShapeDtypeStruct((N, 256), jnp.float32),
    )(lo, hi, den, h_in, bias, g, be)


def kernel(x, edge_index, edge_attr, category, class_tab, index_tab, np_W, np_b, np_g, np_be, ep_W, ep_b, ep_g, ep_be, l0_Wl, l0_bl, l0_Wr, l0_br, l0_We, l0_att, l0_bias, l0_g, l0_be, l1_Wl, l1_bl, l1_Wr, l1_br, l1_We, l1_att, l1_bias, l1_g, l1_be, l2_Wl, l2_bl, l2_Wr, l2_br, l2_We, l2_att, l2_bias, l2_g, l2_be, th_W1, th_b1, th_W2, th_b2, rh_W1, rh_b1, rh_W2, rh_b2):
    src, dst = edge_index[0], edge_index[1]
    cls = class_tab[category]
    bidx = jnp.arange(N) % 50
    idx_emb = index_tab[bidx]
    anchor = jnp.zeros((N, 1), jnp.float32).at[0, 0].set(1.0)
    h0 = jnp.concatenate([cls, x, idx_emb, anchor], axis=1)
    h = _mm(h0, np_W, np_b, np_g, np_be, act="relu")
    ea = _mm(edge_attr, ep_W, ep_b, ep_g, ep_be, act="relu")

    gI, wI, dI = _pad_edges(jnp.arange(E, dtype=jnp.int32),
                            jnp.ones((E, 8), jnp.float32),
                            dst.astype(jnp.int32))
    mout, mden = _sc_scatter(ea, gI, wI, dI)
    deg = jnp.maximum(mden[:N, :1], 1.0)
    ea_mean = jnp.concatenate([mout[0, :N], mout[1, :N]], axis=1) / deg

    loop = jnp.arange(N, dtype=jnp.int32)
    src2 = jnp.concatenate([src.astype(jnp.int32), loop])
    dst2 = jnp.concatenate([dst.astype(jnp.int32), loop])
    ea2 = jnp.concatenate([ea, ea_mean], axis=0)

    layers = [
        (l0_Wl, l0_bl, l0_Wr, l0_br, l0_We, l0_att, l0_bias, l0_g, l0_be),
        (l1_Wl, l1_bl, l1_Wr, l1_br, l1_We, l1_att, l1_bias, l1_g, l1_be),
        (l2_Wl, l2_bl, l2_Wr, l2_br, l2_We, l2_att, l2_bias, l2_g, l2_be),
    ]
    zb = jnp.zeros((HID,), jnp.float32)
    for (Wl, bl, Wr, br, We, att, bias, g, be) in layers:
        e = _mm(ea2, We, zb)
        x_l = _mm(h, Wl, bl)
        x_r = _mm(h, Wr, br)
        m = (x_l.reshape(N, H, FH)[src2] + x_r.reshape(N, H, FH)[dst2]
             + e.reshape(-1, H, FH))
        m = jax.nn.leaky_relu(m, 0.2)
        alpha = jnp.sum(m * att[None], axis=-1)
        ex = jnp.exp(alpha)
        gE, wE, dE = _pad_edges(src2, ex, dst2)
        out2, den2 = _sc_scatter(x_l, gE, wE, dE)
        h = _norm_layer(out2[0, :N], out2[1, :N], den2[:N], h, bias, g, be)

    raw = jnp.tanh(_mm(h, th_W1, th_b1, act="relu") @ th_W2 + th_b2)
    pos = raw * 2.0
    q = _mm(h, rh_W1, rh_b1, act="relu") @ rh_W2 + rh_b2
    q = q / jnp.maximum(jnp.linalg.norm(q, axis=-1, keepdims=True), 1e-12)
    pos = pos.at[0].set(0.0)
    return pos, q


# speedup vs baseline: 9.4698x; 1.7397x over previous
import functools
import math

import jax
import jax.numpy as jnp
from jax import lax
from jax.experimental import pallas as pl
from jax.experimental.pallas import tpu as pltpu
from jax.experimental.pallas import tpu_sc as plsc

N = 10000
E = 160000
H = 8
FH = 32
HID = 256

NSUB = 16
LANES = 16
EB = 128
NP = 10240
NROW = NP // NSUB
ZCH = 128

_SC_MESH = plsc.VectorSubcoreMesh(
    core_axis_name="c", subcore_axis_name="s", num_cores=2, num_subcores=16)


def _mm_kernel(x_ref, w_ref, b_ref, o_ref, *, act, ln, g_ref=None, be_ref=None):
    acc = jnp.dot(x_ref[...], w_ref[...], preferred_element_type=jnp.float32)
    acc = acc + b_ref[...]
    if ln:
        m = jnp.mean(acc, axis=-1, keepdims=True)
        v = jnp.mean((acc - m) ** 2, axis=-1, keepdims=True)
        acc = (acc - m) * lax.rsqrt(v + 1e-5) * g_ref[...] + be_ref[...]
    if act == "relu":
        acc = jnp.maximum(acc, 0.0)
    o_ref[...] = acc


def _mm(x, W, b, g=None, be=None, act=None, tm=512):
    M, K = x.shape
    N_out = W.shape[1]
    Mp = math.ceil(M / tm) * tm
    if Mp != M:
        x = jnp.pad(x, ((0, Mp - M), (0, 0)))
    ln = g is not None
    kern = functools.partial(_mm_kernel, act=act, ln=False)
    in_specs = [
        pl.BlockSpec((tm, K), lambda i: (i, 0)),
        pl.BlockSpec((K, N_out), lambda i: (0, 0)),
        pl.BlockSpec((N_out,), lambda i: (0,)),
    ]
    args = [x, W, b]
    if ln:
        def kern(x_ref, w_ref, b_ref, g_ref, be_ref, o_ref):
            _mm_kernel(x_ref, w_ref, b_ref, o_ref, act=act, ln=True,
                       g_ref=g_ref, be_ref=be_ref)
        in_specs += [
            pl.BlockSpec((N_out,), lambda i: (0,)),
            pl.BlockSpec((N_out,), lambda i: (0,)),
        ]
        args += [g, be]
    out = pl.pallas_call(
        kern,
        grid=(Mp // tm,),
        in_specs=in_specs,
        out_specs=pl.BlockSpec((tm, N_out), lambda i: (i, 0)),
        out_shape=jax.ShapeDtypeStruct((Mp, N_out), jnp.float32),
    )(*args)
    return out[:M]


def _sc_scatter_body(vcat_hbm, gcat_hbm, wsel_hbm, dst_hbm, out_hbm,
                     acc_sh, gidx_v, dst_v, w_v, rows_v, sem,
                     *, nblk):
    c = lax.axis_index("c")
    s = lax.axis_index("s")
    mtot = nblk * EB * NSUB

    def zrow_body(i, _):
        for j8 in range(8):
            rows_v[i, pl.ds(j8 * 16, 16)] = jnp.zeros((16,), jnp.float32)
        return 0
    lax.fori_loop(0, ZCH, zrow_body, 0)

    def zcopy(z, _):
        r0 = s * NROW + z * ZCH
        pltpu.sync_copy(rows_v, acc_sh.at[pl.ds(r0, ZCH)])
        return 0
    lax.fori_loop(0, NROW // ZCH, zcopy, 0)
    plsc.subcore_barrier()

    mw = nblk * EB
    gdn = lax.GatherDimensionNumbers(
        offset_dims=(), collapsed_slice_dims=(0,), start_index_map=(0,))

    def blk_body(b, _):
        base = s * mw + b * EB
        pltpu.sync_copy(gcat_hbm.at[pl.ds(c * mtot + base, EB)], gidx_v)
        pltpu.sync_copy(dst_hbm.at[pl.ds(base, EB)], dst_v)
        pltpu.sync_copy(wsel_hbm.at[pl.ds(c * mtot * 8 + base * 8, EB * 8)], w_v)
        pltpu.async_copy(vcat_hbm.at[gidx_v], rows_v, sem).wait()

        def ebody(i, _):
            wvec = w_v[pl.ds(i * 16, 16)]
            for half in range(2):
                e = i * 2 + half
                for j in range(4):
                    spl = lax.gather(
                        wvec, jnp.full((16, 1), half * 8 + j, jnp.int32), gdn,
                        slice_sizes=(1,),
                        mode=lax.GatherScatterMode.PROMISE_IN_BOUNDS)
                    for t in range(2):
                        v = j * 2 + t
                        rows_v[e, pl.ds(v * 16, 16)] = (
                            rows_v[e, pl.ds(v * 16, 16)] * spl)
            return 0
        lax.fori_loop(0, EB // 2, ebody, 0)

        pltpu.sync_copy(rows_v, acc_sh.at[dst_v], add=True)
        return 0

    lax.fori_loop(0, nblk, blk_body, 0)
    plsc.subcore_barrier()

    def ecopy(z, _):
        r0 = s * NROW + z * ZCH
        pltpu.sync_copy(acc_sh.at[pl.ds(r0, ZCH)], rows_v)
        pltpu.sync_copy(rows_v, out_hbm.at[c, pl.ds(r0, ZCH)])
        return 0
    lax.fori_loop(0, NROW // ZCH, ecopy, 0)


def _sc_den_body(wflat_hbm, dst_hbm, den_hbm, den_sh, dst_v, wtmp_v, w128_v,
                 *, nblk):
    c = lax.axis_index("c")
    s = lax.axis_index("s")

    def zrow_body(i, _):
        for j8 in range(8):
            w128_v[i, pl.ds(j8 * 16, 16)] = jnp.zeros((16,), jnp.float32)
        return 0
    lax.fori_loop(0, EB, zrow_body, 0)

    def zcopy(z, _):
        r0 = s * NROW + z * ZCH
        pltpu.sync_copy(w128_v, den_sh.at[pl.ds(r0, ZCH)])
        return 0
    lax.fori_loop(0, NROW // ZCH, zcopy, 0)
    plsc.subcore_barrier()

    mw = nblk * EB // 2
    gdn = lax.GatherDimensionNumbers(
        offset_dims=(), collapsed_slice_dims=(0,), start_index_map=(0,))
    hi_idx = ((lax.iota(jnp.int32, 16) % 8) + 8).reshape(16, 1)

    def blk_body(b, _):
        base = (s * 2 + c) * mw + b * EB
        pltpu.sync_copy(dst_hbm.at[pl.ds(base, EB)], dst_v)
        pltpu.sync_copy(wflat_hbm.at[pl.ds(base * 8, EB * 8)], wtmp_v)

        def ebody(i, _):
            wvec = wtmp_v[pl.ds(i * 16, 16)]
            w128_v[i * 2, pl.ds(0, 16)] = wvec
            w128_v[i * 2 + 1, pl.ds(0, 16)] = lax.gather(
                wvec, hi_idx, gdn, slice_sizes=(1,),
                mode=lax.GatherScatterMode.PROMISE_IN_BOUNDS)
            return 0
        lax.fori_loop(0, EB // 2, ebody, 0)

        pltpu.sync_copy(w128_v, den_sh.at[dst_v], add=True)
        return 0
    lax.fori_loop(0, nblk // 2, blk_body, 0)
    plsc.subcore_barrier()

    def ecopy(z, _):
        r0 = s * NROW + z * ZCH
        pltpu.sync_copy(den_sh.at[pl.ds(r0, ZCH)], w128_v)
        pltpu.sync_copy(w128_v, den_hbm.at[c, pl.ds(r0, ZCH)])
        return 0
    lax.fori_loop(0, NROW // ZCH, ecopy, 0)


def _sc_scatter(vals, gidx, wh, dstidx):
    M = gidx.shape[0]
    NV = vals.shape[0]
    nblk = M // (NSUB * EB)
    vcat = jnp.concatenate([vals[:, :128], vals[:, 128:]], axis=0)
    gcat = jnp.concatenate([gidx, gidx + NV])
    wsel = jnp.stack(
        [jnp.pad(wh[:, :4], ((0, 0), (0, 4))),
         jnp.pad(wh[:, 4:], ((0, 0), (0, 4)))]).reshape(-1)
    wpk = wh.reshape(-1)
    acc_fn = pl.kernel(
        functools.partial(_sc_scatter_body, nblk=nblk),
        out_type=jax.ShapeDtypeStruct((2, NP, 128), jnp.float32),
        mesh=_SC_MESH,
        scratch_types=[
            pltpu.VMEM_SHARED((NP, 128), jnp.float32),
            pltpu.VMEM((EB,), jnp.int32),
            pltpu.VMEM((EB,), jnp.int32),
            pltpu.VMEM((EB * 8,), jnp.float32),
            pltpu.VMEM((EB, 128), jnp.float32),
            pltpu.SemaphoreType.DMA,
        ],
    )
    den_fn = pl.kernel(
        functools.partial(_sc_den_body, nblk=nblk),
        out_type=jax.ShapeDtypeStruct((2, NP, 128), jnp.float32),
        mesh=_SC_MESH,
        scratch_types=[
            pltpu.VMEM_SHARED((NP, 128), jnp.float32),
            pltpu.VMEM((EB,), jnp.int32),
            pltpu.VMEM((EB * 8,), jnp.float32),
            pltpu.VMEM((EB, 128), jnp.float32),
        ],
    )
    out = acc_fn(vcat, gcat, wsel, dstidx)
    den2 = den_fn(wpk, dstidx)
    return out, den2[0, :, :8] + den2[1, :, :8]


def _sc_alpha_body(xl_hbm, xr_hbm, e2_hbm, src_hbm, dst_hbm, att_hbm, ex_hbm,
                   att_v, src_v, dst_v, xlr_v, xrr_v, er_v, ex_v, sem,
                   *, nblk):
    c = lax.axis_index("c")
    s = lax.axis_index("s")
    pltpu.sync_copy(att_hbm, att_v)
    lane = lax.iota(jnp.int32, 16)
    gdn = lax.GatherDimensionNumbers(
        offset_dims=(), collapsed_slice_dims=(0,), start_index_map=(0,))
    perms = [(lane ^ (1 << k)).reshape(16, 1) for k in range(4)]
    mw = nblk * EB // 2

    def blk_body(b, _):
        base = (s * 2 + c) * mw + b * EB
        pltpu.sync_copy(src_hbm.at[pl.ds(base, EB)], src_v)
        pltpu.sync_copy(dst_hbm.at[pl.ds(base, EB)], dst_v)
        pltpu.sync_copy(e2_hbm.at[pl.ds(base, EB)], er_v)
        pltpu.async_copy(xl_hbm.at[src_v], xlr_v, sem).wait()
        pltpu.async_copy(xr_hbm.at[dst_v], xrr_v, sem).wait()

        def ebody(i, _):
            r = jnp.zeros((16,), jnp.float32)
            for half in range(2):
                e = i * 2 + half
                for h in range(8):
                    tot = None
                    t = None
                    for sub in range(2):
                        v = h * 2 + sub
                        m = (xlr_v[e, pl.ds(v * 16, 16)]
                             + xrr_v[e, pl.ds(v * 16, 16)]
                             + er_v[e, pl.ds(v * 16, 16)])
                        m = jnp.maximum(m, 0.2 * m)
                        tt = m * att_v[pl.ds(v * 16, 16)]
                        t = tt if t is None else t + tt
                    tot = t
                    for p in perms:
                        tot = tot + lax.gather(
                            tot, p, gdn, slice_sizes=(1,),
                            mode=lax.GatherScatterMode.PROMISE_IN_BOUNDS)
                    r = jnp.where(lane == half * 8 + h, tot, r)
            ex_v[pl.ds(i * 16, 16)] = jnp.exp(r)
            return 0
        lax.fori_loop(0, EB // 2, ebody, 0)

        pltpu.sync_copy(ex_v, ex_hbm.at[pl.ds(base * 8, EB * 8)])
        return 0
    lax.fori_loop(0, nblk // 2, blk_body, 0)


def _sc_alpha(xl, xr, e2p, srcp, dstp, att_flat):
    Mp = srcp.shape[0]
    nblk = Mp // (NSUB * EB)
    fn = pl.kernel(
        functools.partial(_sc_alpha_body, nblk=nblk),
        out_type=jax.ShapeDtypeStruct((Mp * 8,), jnp.float32),
        mesh=_SC_MESH,
        scratch_types=[
            pltpu.VMEM((256,), jnp.float32),
            pltpu.VMEM((EB,), jnp.int32),
            pltpu.VMEM((EB,), jnp.int32),
            pltpu.VMEM((EB, 256), jnp.float32),
            pltpu.VMEM((EB, 256), jnp.float32),
            pltpu.VMEM((EB, 256), jnp.float32),
            pltpu.VMEM((EB * 8,), jnp.float32),
            pltpu.SemaphoreType.DMA,
        ],
    )
    return fn(xl, xr, e2p, srcp, dstp, att_flat).reshape(Mp, 8)


def _pad_edges(gidx, wh, dstidx):
    M = gidx.shape[0]
    Mp = math.ceil(M / (2 * NSUB * EB)) * (2 * NSUB * EB)
    pad = Mp - M
    gidx = jnp.pad(gidx, (0, pad))
    dstidx = jnp.pad(dstidx, (0, pad))
    wh = jnp.pad(wh, ((0, pad), (0, 0)))
    return gidx, wh, dstidx


def _norm_kernel(lo_ref, hi_ref, den_ref, hin_ref, bias_ref, g_ref, be_ref, o_ref):
    den = den_ref[...]
    tm = den.shape[0]
    den = jnp.broadcast_to(den[:, :, None], (tm, 8, 32)).reshape(tm, 256)
    unnorm = jnp.concatenate([lo_ref[...], hi_ref[...]], axis=1)
    h = unnorm / (den + 1e-16) + bias_ref[...] + hin_ref[...]
    m = jnp.mean(h, axis=-1, keepdims=True)
    v = jnp.mean((h - m) ** 2, axis=-1, keepdims=True)
    h = (h - m) * lax.rsqrt(v + 1e-5) * g_ref[...] + be_ref[...]
    o_ref[...] = jnp.where(h > 0, h, jnp.exp(h) - 1.0)


def _norm_layer(lo, hi, den, h_in, bias, g, be, tm=400):
    grid = (N // tm,)
    return pl.pallas_call(
        _norm_kernel,
        grid=grid,
        in_specs=[
            pl.BlockSpec((tm, 128), lambda i: (i, 0)),
            pl.BlockSpec((tm, 128), lambda i: (i, 0)),
            pl.BlockSpec((tm, 8), lambda i: (i, 0)),
            pl.BlockSpec((tm, 256), lambda i: (i, 0)),
            pl.BlockSpec((256,), lambda i: (0,)),
            pl.BlockSpec((256,), lambda i: (0,)),
            pl.BlockSpec((256,), lambda i: (0,)),
        ],
        out_specs=pl.BlockSpec((tm, 256), lambda i: (i, 0)),
        out_shape=jax.ShapeDtypeStruct((N, 256), jnp.float32),
    )(lo, hi, den, h_in, bias, g, be)


def kernel(x, edge_index, edge_attr, category, class_tab, index_tab, np_W, np_b, np_g, np_be, ep_W, ep_b, ep_g, ep_be, l0_Wl, l0_bl, l0_Wr, l0_br, l0_We, l0_att, l0_bias, l0_g, l0_be, l1_Wl, l1_bl, l1_Wr, l1_br, l1_We, l1_att, l1_bias, l1_g, l1_be, l2_Wl, l2_bl, l2_Wr, l2_br, l2_We, l2_att, l2_bias, l2_g, l2_be, th_W1, th_b1, th_W2, th_b2, rh_W1, rh_b1, rh_W2, rh_b2):
    src, dst = edge_index[0], edge_index[1]
    cls = class_tab[category]
    bidx = jnp.arange(N) % 50
    idx_emb = index_tab[bidx]
    anchor = jnp.zeros((N, 1), jnp.float32).at[0, 0].set(1.0)
    h0 = jnp.concatenate([cls, x, idx_emb, anchor], axis=1)
    h = _mm(h0, np_W, np_b, np_g, np_be, act="relu")
    ea = _mm(edge_attr, ep_W, ep_b, ep_g, ep_be, act="relu")

    gI, wI, dI = _pad_edges(jnp.arange(E, dtype=jnp.int32),
                            jnp.ones((E, 8), jnp.float32),
                            dst.astype(jnp.int32))
    mout, mden = _sc_scatter(ea, gI, wI, dI)
    deg = jnp.maximum(mden[:N, :1], 1.0)
    ea_mean = jnp.concatenate([mout[0, :N], mout[1, :N]], axis=1) / deg

    loop = jnp.arange(N, dtype=jnp.int32)
    src2 = jnp.concatenate([src.astype(jnp.int32), loop])
    dst2 = jnp.concatenate([dst.astype(jnp.int32), loop])
    ea2 = jnp.concatenate([ea, ea_mean], axis=0)
    M2 = E + N
    Mp = math.ceil(M2 / (2 * NSUB * EB)) * (2 * NSUB * EB)
    srcp = jnp.pad(src2, (0, Mp - M2))
    dstp = jnp.pad(dst2, (0, Mp - M2), constant_values=N)
    epad = Mp - M2

    layers = [
        (l0_Wl, l0_bl, l0_Wr, l0_br, l0_We, l0_att, l0_bias, l0_g, l0_be),
        (l1_Wl, l1_bl, l1_Wr, l1_br, l1_We, l1_att, l1_bias, l1_g, l1_be),
        (l2_Wl, l2_bl, l2_Wr, l2_br, l2_We, l2_att, l2_bias, l2_g, l2_be),
    ]
    zb = jnp.zeros((HID,), jnp.float32)
    for (Wl, bl, Wr, br, We, att, bias, g, be) in layers:
        e = _mm(ea2, We, zb)
        x_l = _mm(h, Wl, bl)
        x_r = _mm(h, Wr, br)
        e2p = jnp.pad(e, ((0, epad), (0, 0)))
        xrp = jnp.pad(x_r, ((0, NP - N), (0, 0)))
        ex = _sc_alpha(x_l, xrp, e2p, srcp, dstp, att.reshape(-1))
        out2, den2 = _sc_scatter(x_l, srcp, ex, dstp)
        h = _norm_layer(out2[0, :N], out2[1, :N], den2[:N], h, bias, g, be)

    raw = jnp.tanh(_mm(h, th_W1, th_b1, act="relu") @ th_W2 + th_b2)
    pos = raw * 2.0
    q = _mm(h, rh_W1, rh_b1, act="relu") @ rh_W2 + rh_b2
    q = q / jnp.maximum(jnp.linalg.norm(q, axis=-1, keepdims=True), 1e-12)
    pos = pos.at[0].set(0.0)
    return pos, q
